# baseline (speedup 1.0000x reference)
import functools

import jax
import jax.numpy as jnp
from jax import lax
from jax.experimental import pallas as pl
from jax.experimental.pallas import tpu as pltpu
from jax.experimental.pallas import tpu_sc as plsc

_P = 0.15
_B, _F = 16384, 100
_N = _B * _F
_NC, _NS = 2, 16
_NW = _NC * _NS
_NPW = _N // _NW
_CHUNK = 10240
_NCHUNK = _NPW // _CHUNK
_G = 128
_NG = _CHUNK // _G
_VPG = _G // 16
_INFLIGHT = 8


@functools.partial(
    pl.kernel,
    out_type=jax.ShapeDtypeStruct((_N // _G, _G), jnp.float32),
    mesh=plsc.VectorSubcoreMesh(core_axis_name="c", subcore_axis_name="s"),
    scratch_types=[
        pltpu.VMEM((_CHUNK,), jnp.float32),
        pltpu.VMEM((_CHUNK,), jnp.float32),
        pltpu.VMEM((_NG, _G), jnp.int32),
        pltpu.VMEM((_NG, _G), jnp.float32),
        pltpu.SemaphoreType.DMA,
    ],
)
def _swap_noise(x_hbm, rm_hbm, rr_hbm, out_hbm, rm_v, rr_v, idx_v, gout_v, sem):
    wid = lax.axis_index("s") * _NC + lax.axis_index("c")
    base = wid * _NPW
    iota = lax.iota(jnp.int32, 16)
    thresh = jnp.float32(1.0 - _P)
    bf = jnp.float32(_B)

    def chunk_body(c, carry):
        off = pl.multiple_of(base + c * _CHUNK, _CHUNK)
        pltpu.sync_copy(rm_hbm.at[pl.ds(off, _CHUNK)], rm_v)
        pltpu.sync_copy(rr_hbm.at[pl.ds(off, _CHUNK)], rr_v)

        def vec_body(g, carry2):
            def lane_body(u, carry3):
                p = g * _G + u * 16
                rm = rm_v[pl.ds(p, 16)]
                rr = rr_v[pl.ds(p, 16)]
                k = (off + p) + iota
                s = (rr * bf).astype(jnp.int32) * _F
                idx = k + jnp.where(rm > thresh, s, 0)
                idx = jnp.where(idx >= _N, idx - _N, idx)
                idx_v[g, pl.ds(u * 16, 16)] = idx
                return carry3

            return lax.fori_loop(0, _VPG, lane_body, carry2)

        lax.fori_loop(0, _NG, vec_body, 0)

        def gather_group(g0, carry2):
            copies = []
            for u in range(_INFLIGHT):
                g = g0 * _INFLIGHT + u
                copies.append(
                    pltpu.async_copy(x_hbm.at[idx_v.at[g]], gout_v.at[g], sem))
            for cp in copies:
                cp.wait()
            return carry2

        lax.fori_loop(0, _NG // _INFLIGHT, gather_group, 0)

        row0 = pl.multiple_of(off // _G, _NG)
        pltpu.sync_copy(gout_v, out_hbm.at[pl.ds(row0, _NG)])
        return carry

    lax.fori_loop(0, _NCHUNK, chunk_body, 0)


def kernel(x, rand_mask, rand_rows):
    out2d = _swap_noise(
        x.reshape(-1), rand_mask.reshape(-1), rand_rows.reshape(-1))
    return out2d.reshape(_B, _F)


# speedup vs baseline: 1.4489x; 1.4489x over previous
import functools

import jax
import jax.numpy as jnp
from jax import lax
from jax.experimental import pallas as pl
from jax.experimental.pallas import tpu as pltpu
from jax.experimental.pallas import tpu_sc as plsc

_P = 0.15
_B, _F = 16384, 100
_N = _B * _F
_NC, _NS = 2, 16
_NW = _NC * _NS
_NPW = _N // _NW
_CHUNK = 12800
_NCHUNK = _NPW // _CHUNK
_G = 128
_NG = _CHUNK // _G


@functools.partial(
    pl.kernel,
    out_type=jax.ShapeDtypeStruct((_N,), jnp.float32),
    mesh=plsc.VectorSubcoreMesh(core_axis_name="c", subcore_axis_name="s"),
    compiler_params=pltpu.CompilerParams(needs_layout_passes=False),
    scratch_types=[
        pltpu.VMEM((_CHUNK,), jnp.float32),
        pltpu.VMEM((_CHUNK,), jnp.float32),
        pltpu.VMEM((_NG + 1, _G), jnp.int32),
        pltpu.VMEM((_CHUNK + _G,), jnp.int32),
        pltpu.VMEM((_NG, _G), jnp.float32),
        pltpu.VMEM((_CHUNK + 16,), jnp.float32),
        pltpu.VMEM((16,), jnp.int32),
        pltpu.SemaphoreType.DMA,
        pltpu.SemaphoreType.DMA,
    ],
)
def _swap_noise(x_hbm, rm_hbm, rr_hbm, out_hbm,
                rm_v, rr_v, gidx_v, pos_v, gath_v, out_v, cnt_v,
                sem, sem2):
    wid = lax.axis_index("s") * _NC + lax.axis_index("c")
    base = wid * _NPW
    iota = lax.iota(jnp.int32, 16)
    thresh = jnp.float32(1.0 - _P)
    bf = jnp.float32(_B)
    zeros_i = jnp.zeros((16,), jnp.int32)

    def chunk_body(c, carry):
        off = pl.multiple_of(base + c * _CHUNK, _CHUNK)
        idcp = pltpu.async_copy(
            x_hbm.at[pl.ds(off, _CHUNK)], out_v.at[pl.ds(0, _CHUNK)], sem2)
        pltpu.sync_copy(rm_hbm.at[pl.ds(off, _CHUNK)], rm_v)
        pltpu.sync_copy(rr_hbm.at[pl.ds(off, _CHUNK)], rr_v)

        def vec_body(v, acc):
            p = v * 16
            rm = rm_v[pl.ds(p, 16)]
            rr = rr_v[pl.ds(p, 16)]
            lanep = p + iota
            m = rm > thresh
            s = (rr * bf).astype(jnp.int32) * _F
            idx = (off + lanep) + s
            idx = jnp.where(idx >= _N, idx - _N, idx)
            mi = jnp.where(m, 1, 0)
            dst = acc + (plsc.cumsum(mi) - mi)
            plsc.store_scatter(gidx_v, [dst >> 7, dst & 127], idx, mask=m)
            plsc.store_scatter(pos_v, [dst], lanep, mask=m)
            return acc + plsc.all_reduce_population_count(m)

        acc = lax.fori_loop(0, _CHUNK // 16, vec_body, zeros_i)
        cnt = acc[0]

        gbase = (cnt >> 7) << 7
        for u in range(8):
            lp = (gbase + u * 16) + iota
            tm = lp >= cnt
            plsc.store_scatter(gidx_v, [lp >> 7, lp & 127], zeros_i, mask=tm)
            plsc.store_scatter(pos_v, [lp], _CHUNK + iota, mask=tm)

        ng = (cnt + 127) >> 7

        def fire(g, carry2):
            pltpu.async_copy(x_hbm.at[gidx_v.at[g]], gath_v.at[g], sem)
            return carry2

        lax.fori_loop(0, ng, fire, 0)
        idcp.wait()

        def drain(g, carry2):
            pltpu.make_async_copy(
                x_hbm.at[gidx_v.at[g]], gath_v.at[g], sem).wait()
            return carry2

        lax.fori_loop(0, ng, drain, 0)

        def scatter_back(g, carry2):
            for u in range(8):
                val = gath_v[g, pl.ds(u * 16, 16)]
                pvec = pos_v[pl.ds(g * _G + u * 16, 16)]
                plsc.store_scatter(out_v, [pvec], val)
            return carry2

        lax.fori_loop(0, ng, scatter_back, 0)
        pltpu.sync_copy(out_v.at[pl.ds(0, _CHUNK)], out_hbm.at[pl.ds(off, _CHUNK)])
        return carry

    lax.fori_loop(0, _NCHUNK, chunk_body, 0)


def kernel(x, rand_mask, rand_rows):
    out = _swap_noise(
        x.reshape(-1), rand_mask.reshape(-1), rand_rows.reshape(-1))
    return out.reshape(_B, _F)


# speedup vs baseline: 1.6784x; 1.1584x over previous
import functools

import jax
import jax.numpy as jnp
from jax import lax
from jax.experimental import pallas as pl
from jax.experimental.pallas import tpu as pltpu
from jax.experimental.pallas import tpu_sc as plsc

_P = 0.15
_B, _F = 16384, 100
_N = _B * _F
_NC, _NS = 2, 16
_NW = _NC * _NS
_RPW = _B // _NW
_CH_R = 128
_NCHUNK = _RPW // _CH_R
_CHUNK = _CH_R * _F
_G = 128
_NG = _CHUNK // _G
_VPR = 7


@functools.partial(
    pl.kernel,
    out_type=jax.ShapeDtypeStruct((_B, _F), jnp.float32),
    mesh=plsc.VectorSubcoreMesh(core_axis_name="c", subcore_axis_name="s"),
    compiler_params=pltpu.CompilerParams(
        needs_layout_passes=False, use_tc_tiling_on_sc=True),
    scratch_types=[
        pltpu.VMEM((_CH_R, _F), jnp.float32),
        pltpu.VMEM((_CH_R, _F), jnp.float32),
        pltpu.VMEM((_CH_R + 8, _F), jnp.float32),
        pltpu.VMEM((_NG + 1, _G), jnp.int32),
        pltpu.VMEM((_CHUNK + _G,), jnp.int32),
        pltpu.VMEM((_NG, _G), jnp.float32),
        pltpu.SemaphoreType.DMA,
        pltpu.SemaphoreType.DMA,
    ],
)
def _swap_noise(xf_hbm, x_hbm, rm_hbm, rr_hbm, out_hbm,
                rm_v, rr_v, out_v, gidx_v, pos_v, gath_v, sem, sem2):
    wid = lax.axis_index("s") * _NC + lax.axis_index("c")
    rbase = wid * _RPW
    iota = lax.iota(jnp.int32, 16)
    thresh = jnp.float32(1.0 - _P)
    bf = jnp.float32(_B)
    zeros_i = jnp.zeros((16,), jnp.int32)
    valid6 = iota >= (6 * 16 - (_F - 16))

    def chunk_body(c, carry):
        row0 = pl.multiple_of(rbase + c * _CH_R, _CH_R)
        idcp = pltpu.async_copy(
            x_hbm.at[pl.ds(row0, _CH_R)], out_v.at[pl.ds(0, _CH_R)], sem2)
        pltpu.sync_copy(rm_hbm.at[pl.ds(row0, _CH_R)], rm_v)
        pltpu.sync_copy(rr_hbm.at[pl.ds(row0, _CH_R)], rr_v)

        def row_body(r, acc):
            kbase = (row0 + r) * _F
            pbase = r * 128
            for u in range(_VPR):
                c0 = u * 16 if u < _VPR - 1 else _F - 16
                cols = c0 + iota
                rm = rm_v[r, pl.ds(c0, 16)]
                rr = rr_v[r, pl.ds(c0, 16)]
                m = rm > thresh
                if u == _VPR - 1:
                    m = jnp.logical_and(m, valid6)
                s = (rr * bf).astype(jnp.int32) * _F
                idx = (kbase + cols) + s
                idx = jnp.where(idx >= _N, idx - _N, idx)
                mi = jnp.where(m, 1, 0)
                dst = acc + (plsc.cumsum(mi) - mi)
                plsc.store_scatter(gidx_v, [dst >> 7, dst & 127], idx, mask=m)
                plsc.store_scatter(pos_v, [dst], pbase + cols, mask=m)
                acc = acc + plsc.all_reduce_population_count(m)
            return acc

        acc = lax.fori_loop(0, _CH_R, row_body, zeros_i)
        cnt = acc[0]

        gbase = (cnt >> 7) << 7
        for u in range(8):
            lp = (gbase + u * 16) + iota
            tm = lp >= cnt
            plsc.store_scatter(gidx_v, [lp >> 7, lp & 127], zeros_i, mask=tm)
            plsc.store_scatter(pos_v, [lp], (_CH_R << 7) + iota, mask=tm)

        ng = (cnt + 127) >> 7

        def fire(g, carry2):
            pltpu.async_copy(xf_hbm.at[gidx_v.at[g]], gath_v.at[g], sem)
            return carry2

        lax.fori_loop(0, ng, fire, 0)
        idcp.wait()

        def drain(g, carry2):
            pltpu.make_async_copy(
                xf_hbm.at[gidx_v.at[g]], gath_v.at[g], sem).wait()
            return carry2

        lax.fori_loop(0, ng, drain, 0)

        def scatter_back(g, carry2):
            for u in range(8):
                val = gath_v[g, pl.ds(u * 16, 16)]
                pvec = pos_v[pl.ds(g * _G + u * 16, 16)]
                plsc.store_scatter(out_v, [pvec >> 7, pvec & 127], val)
            return carry2

        lax.fori_loop(0, ng, scatter_back, 0)
        pltpu.sync_copy(out_v.at[pl.ds(0, _CH_R)], out_hbm.at[pl.ds(row0, _CH_R)])
        return carry

    lax.fori_loop(0, _NCHUNK, chunk_body, 0)


def kernel(x, rand_mask, rand_rows):
    return _swap_noise(x.reshape(-1), x, rand_mask, rand_rows)
